# baseline (speedup 1.0000x reference)
import functools

import jax
import jax.numpy as jnp
from jax import lax
from jax.experimental import pallas as pl
from jax.experimental.pallas import tpu as pltpu
from jax.experimental.pallas import tpu_sc as plsc

NW = 32
NB = 8
LANES = 16


def _make_kernel(B, L, V, D):
    PW = L // NW
    CHUNKS = B // NB
    ROWS = NB * PW

    mesh = plsc.VectorSubcoreMesh(core_axis_name="c", subcore_axis_name="s")

    @functools.partial(
        pl.kernel,
        mesh=mesh,
        out_type=jax.ShapeDtypeStruct((B, L, D), jnp.float32),
        scratch_types=[
            pltpu.VMEM((CHUNKS, ROWS), jnp.int32),
            pltpu.VMEM((PW, D), jnp.float32),
            pltpu.VMEM((ROWS, D), jnp.float32),
            pltpu.SemaphoreType.DMA,
        ],
    )
    def sc_kernel(z_r_hbm, pos_hbm, table_hbm, out_hbm, idx_v, pos_v, rows_v, sem):
        w = lax.axis_index("s") * 2 + lax.axis_index("c")
        pltpu.sync_copy(z_r_hbm.at[w], idx_v)
        pltpu.sync_copy(pos_hbm.at[pl.ds(w * PW, PW)], pos_v)

        def chunk_body(c, carry):
            pltpu.async_copy(table_hbm.at[idx_v.at[c]], rows_v, sem).wait()

            def batch_body(i, carry2):
                for j in range(PW):
                    r = i * PW + j
                    for k in range(D // LANES):
                        sl = pl.ds(k * LANES, LANES)
                        rows_v[r, sl] = rows_v[r, sl] + pos_v[j, sl]
                return carry2

            lax.fori_loop(0, NB, batch_body, 0)

            for i in range(NB):
                b = c * NB + i
                pltpu.sync_copy(
                    rows_v.at[pl.ds(i * PW, PW)],
                    out_hbm.at[b, pl.ds(w * PW, PW), :],
                )
            return carry

        lax.fori_loop(0, CHUNKS, chunk_body, 0)

    return sc_kernel


def kernel(z, latent_embed, pos_embed):
    B, L = z.shape
    V, D = latent_embed.shape
    PW = L // NW
    z_r = (
        z.astype(jnp.int32)
        .reshape(B // NB, NB, NW, PW)
        .transpose(2, 0, 1, 3)
        .reshape(NW, B // NB, NB * PW)
    )
    pos = pos_embed.reshape(L, D)
    return _make_kernel(B, L, V, D)(z_r, pos, latent_embed)
